# baseline (speedup 1.0000x reference)
import jax
import jax.numpy as jnp
from jax.experimental import pallas as pl


def _copy_block(src_ref, dst_ref):
    dst_ref[...] = src_ref[...]


def _pallas_copy(x, block_rows):
    rows, cols = x.shape
    grid = (rows // block_rows,)
    return pl.pallas_call(
        _copy_block,
        grid=grid,
        in_specs=[pl.BlockSpec((block_rows, cols), lambda i: (i, 0))],
        out_specs=pl.BlockSpec((block_rows, cols), lambda i: (i, 0)),
        out_shape=jax.ShapeDtypeStruct((rows, cols), x.dtype),
    )(x)


def kernel(user_w, item_w, edge_index):
    del edge_index
    user_out = _pallas_copy(user_w, block_rows=5000)
    item_out = _pallas_copy(item_w, block_rows=20000)
    return (user_out, item_out)
